# baseline (speedup 1.0000x reference)
import jax
import jax.numpy as jnp
from jax.experimental import pallas as pl

_TILE = 2048


def _mlp_kernel(x_ref, w1_ref, b1_ref, w2_ref, b2_ref,
                wa1_ref, ba1_ref, wa2_ref, ba2_ref,
                wc1_ref, bc1_ref, wc2_ref, bc2_ref,
                logits_ref, value_ref):
    n_act = wa2_ref.shape[1]
    x = x_ref[...]
    h = jnp.maximum(
        jnp.dot(x, w1_ref[...], preferred_element_type=jnp.float32) + b1_ref[...], 0.0)
    e = jnp.maximum(
        jnp.dot(h, w2_ref[...], preferred_element_type=jnp.float32) + b2_ref[...], 0.0)
    wh1 = jnp.concatenate([wa1_ref[...], wc1_ref[...]], axis=1)
    bh1 = jnp.concatenate([ba1_ref[...], bc1_ref[...]], axis=1)
    ac = jnp.maximum(
        jnp.dot(e, wh1, preferred_element_type=jnp.float32) + bh1, 0.0)
    half = wa1_ref.shape[0]
    wh2 = jnp.concatenate([
        jnp.concatenate([wa2_ref[...], jnp.zeros((half, 1), jnp.float32)], axis=1),
        jnp.concatenate([jnp.zeros((half, n_act), jnp.float32), wc2_ref[...]], axis=1),
    ], axis=0)
    bh2 = jnp.concatenate([ba2_ref[...], bc2_ref[...]], axis=1)
    out = jnp.dot(ac, wh2, preferred_element_type=jnp.float32) + bh2
    logits_ref[...] = out[:, :n_act]
    value_ref[...] = out[:, n_act:n_act + 1]


def kernel(global_state, W1, b1, W2, b2, Wa1, ba1, Wa2, ba2, Wc1, bc1, Wc2, bc2):
    B, in_dim = global_state.shape
    n_act = Wa2.shape[1]
    grid = (B // _TILE,)

    def row_block(n):
        return pl.BlockSpec((_TILE, n), lambda i: (i, 0))

    def whole(a):
        return pl.BlockSpec(a.shape, lambda i: (0,) * a.ndim)

    b1r, b2r = b1[None, :], b2[None, :]
    ba1r, ba2r = ba1[None, :], ba2[None, :]
    bc1r, bc2r = bc1[None, :], bc2[None, :]

    logits, value = pl.pallas_call(
        _mlp_kernel,
        grid=grid,
        in_specs=[
            row_block(in_dim),
            whole(W1), whole(b1r), whole(W2), whole(b2r),
            whole(Wa1), whole(ba1r), whole(Wa2), whole(ba2r),
            whole(Wc1), whole(bc1r), whole(Wc2), whole(bc2r),
        ],
        out_specs=[row_block(n_act), row_block(1)],
        out_shape=[
            jax.ShapeDtypeStruct((B, n_act), jnp.float32),
            jax.ShapeDtypeStruct((B, 1), jnp.float32),
        ],
    )(global_state, W1, b1r, W2, b2r, Wa1, ba1r, Wa2, ba2r, Wc1, bc1r, Wc2, bc2r)
    return (logits, value)


# speedup vs baseline: 1.0765x; 1.0765x over previous
import jax
import jax.numpy as jnp
from jax.experimental import pallas as pl

_TILE = 4096


def _mlp_kernel(x_ref, w1_ref, b1_ref, w2_ref, b2_ref,
                wa1_ref, ba1_ref, wa2_ref, ba2_ref,
                wc1_ref, bc1_ref, wc2_ref, bc2_ref,
                logits_ref, value_ref):
    n_act = wa2_ref.shape[1]
    x = x_ref[...]
    h = jnp.maximum(
        jnp.dot(x, w1_ref[...], preferred_element_type=jnp.float32) + b1_ref[...], 0.0)
    e = jnp.maximum(
        jnp.dot(h, w2_ref[...], preferred_element_type=jnp.float32) + b2_ref[...], 0.0)
    wh1 = jnp.concatenate([wa1_ref[...], wc1_ref[...]], axis=1)
    bh1 = jnp.concatenate([ba1_ref[...], bc1_ref[...]], axis=1)
    ac = jnp.maximum(
        jnp.dot(e, wh1, preferred_element_type=jnp.float32) + bh1, 0.0)
    half = wa1_ref.shape[0]
    wh2 = jnp.concatenate([
        jnp.concatenate([wa2_ref[...], jnp.zeros((half, 1), jnp.float32)], axis=1),
        jnp.concatenate([jnp.zeros((half, n_act), jnp.float32), wc2_ref[...]], axis=1),
    ], axis=0)
    bh2 = jnp.concatenate([ba2_ref[...], bc2_ref[...]], axis=1)
    out = jnp.dot(ac, wh2, preferred_element_type=jnp.float32) + bh2
    logits_ref[...] = out[:, :n_act]
    value_ref[...] = out[:, n_act:n_act + 1]


def kernel(global_state, W1, b1, W2, b2, Wa1, ba1, Wa2, ba2, Wc1, bc1, Wc2, bc2):
    B, in_dim = global_state.shape
    n_act = Wa2.shape[1]
    grid = (B // _TILE,)

    def row_block(n):
        return pl.BlockSpec((_TILE, n), lambda i: (i, 0))

    def whole(a):
        return pl.BlockSpec(a.shape, lambda i: (0,) * a.ndim)

    b1r, b2r = b1[None, :], b2[None, :]
    ba1r, ba2r = ba1[None, :], ba2[None, :]
    bc1r, bc2r = bc1[None, :], bc2[None, :]

    logits, value = pl.pallas_call(
        _mlp_kernel,
        grid=grid,
        in_specs=[
            row_block(in_dim),
            whole(W1), whole(b1r), whole(W2), whole(b2r),
            whole(Wa1), whole(ba1r), whole(Wa2), whole(ba2r),
            whole(Wc1), whole(bc1r), whole(Wc2), whole(bc2r),
        ],
        out_specs=[row_block(n_act), row_block(1)],
        out_shape=[
            jax.ShapeDtypeStruct((B, n_act), jnp.float32),
            jax.ShapeDtypeStruct((B, 1), jnp.float32),
        ],
    )(global_state, W1, b1r, W2, b2r, Wa1, ba1r, Wa2, ba2r, Wc1, bc1r, Wc2, bc2r)
    return (logits, value)
